# baseline (speedup 1.0000x reference)
import functools

import jax
import jax.numpy as jnp
from jax import lax
from jax.experimental import pallas as pl
from jax.experimental.pallas import tpu as pltpu
from jax.experimental.pallas import tpu_sc as plsc

NC = 2
NS = 16
NW = NC * NS

CHUNK = 128
CHUNKS_PER_GRP = 2
GRP = CHUNK * CHUNKS_PER_GRP


def _make_gather(B, D2):
    b_per_w = B // NW
    chunks_per_w = b_per_w // CHUNK
    grps = chunks_per_w // CHUNKS_PER_GRP
    mesh = plsc.VectorSubcoreMesh(
        core_axis_name="c", subcore_axis_name="s",
        num_cores=NC, num_subcores=NS)

    @functools.partial(
        pl.kernel,
        mesh=mesh,
        out_type=jax.ShapeDtypeStruct((B, D2), jnp.float32),
        scratch_types=[
            pltpu.VMEM((chunks_per_w, CHUNK), jnp.int32),
            pltpu.VMEM((2, GRP, D2), jnp.float32),
            pltpu.SemaphoreType.DMA,
        ],
    )
    def gather_kernel(pidx_hbm, tpack_hbm, x_hbm, idx_v, rows_v, gsem):
        wid = lax.axis_index("s") * NC + lax.axis_index("c")
        base_row = wid * b_per_w
        pltpu.sync_copy(pidx_hbm.at[wid], idx_v)

        def body(g, carry):
            buf = lax.rem(g, 2)
            copies = []
            for c in range(CHUNKS_PER_GRP):
                cp = pltpu.async_copy(
                    tpack_hbm.at[idx_v.at[g * CHUNKS_PER_GRP + c]],
                    rows_v.at[buf, pl.ds(c * CHUNK, CHUNK)],
                    gsem)
                copies.append(cp)
            for cp in copies:
                cp.wait()
            pltpu.sync_copy(rows_v.at[buf],
                            x_hbm.at[pl.ds(base_row + g * GRP, GRP)])
            return carry

        lax.fori_loop(0, grps, body, 0)

    return gather_kernel


def _project(x, p, W, b2):
    B = x.shape[0]
    D = W.shape[0]
    H = W.shape[1]
    R = 2048
    grid = (B // R,)

    def body(x_ref, p_ref, w_ref, b_ref, o_ref):
        x = x_ref[...]
        pc = p_ref[...][:, None]
        xb = x[:, :D] + pc * (x[:, D:] - x[:, :D])
        o_ref[...] = jnp.dot(xb, w_ref[...],
                             preferred_element_type=jnp.float32) + b_ref[...]

    return pl.pallas_call(
        body,
        grid=grid,
        in_specs=[
            pl.BlockSpec((R, 2 * D), lambda i: (i, 0)),
            pl.BlockSpec((R,), lambda i: (i,)),
            pl.BlockSpec((D, H), lambda i: (0, 0)),
            pl.BlockSpec((1, H), lambda i: (0, 0)),
        ],
        out_specs=pl.BlockSpec((R, H), lambda i: (i, 0)),
        out_shape=jax.ShapeDtypeStruct((B, H), jnp.float32),
    )(x, p, W, b2)


def kernel(sent1, sent2, table, W, b):
    batch, seq = sent1.shape
    V, D = table.shape
    H = W.shape[1]
    idx = jnp.concatenate([sent1.reshape(-1), sent2.reshape(-1)])
    B = idx.shape[0]
    pidx = (idx >> 1).reshape(NW, B // (NW * CHUNK), CHUNK)
    par = (idx & 1).astype(jnp.float32)
    tpack = table.reshape(V // 2, 2 * D)
    x = _make_gather(B, 2 * D)(pidx, tpack)
    out = _project(x, par, W, b.reshape(1, H))
    out = out.reshape(2, batch, seq, H)
    return out[0], out[1]


# speedup vs baseline: 3.7851x; 3.7851x over previous
import functools

import jax
import jax.numpy as jnp
from jax import lax
from jax.experimental import pallas as pl
from jax.experimental.pallas import tpu as pltpu
from jax.experimental.pallas import tpu_sc as plsc

NC = 2
NS = 16
NW = NC * NS

CHUNK = 128
CHUNKS_PER_GRP = 2
GRP = CHUNK * CHUNKS_PER_GRP

PROJ_BLK = 16384


def _project_table(tT, W, b2):
    D, V = tT.shape
    H = W.shape[1]
    grid = (pl.cdiv(V, PROJ_BLK),)

    def body(t_ref, w_ref, b_ref, p_ref):
        p_ref[...] = lax.dot_general(
            t_ref[...], w_ref[...],
            dimension_numbers=(((0,), (0,)), ((), ())),
            preferred_element_type=jnp.float32) + b_ref[...]

    return pl.pallas_call(
        body,
        grid=grid,
        in_specs=[
            pl.BlockSpec((D, PROJ_BLK), lambda i: (0, i)),
            pl.BlockSpec(W.shape, lambda i: (0, 0)),
            pl.BlockSpec((1, H), lambda i: (0, 0)),
        ],
        out_specs=pl.BlockSpec((PROJ_BLK, H), lambda i: (i, 0)),
        out_shape=jax.ShapeDtypeStruct((V, H), jnp.float32),
    )(tT, W, b2)


def _make_gather(B, H):
    b_per_w = B // NW
    chunks_per_w = b_per_w // CHUNK
    grps = chunks_per_w // CHUNKS_PER_GRP
    half = NS // 2
    mesh = plsc.VectorSubcoreMesh(
        core_axis_name="c", subcore_axis_name="s",
        num_cores=NC, num_subcores=NS)

    @functools.partial(
        pl.kernel,
        mesh=mesh,
        out_type=(
            jax.ShapeDtypeStruct((B // 2, H), jnp.float32),
            jax.ShapeDtypeStruct((B // 2, H), jnp.float32),
        ),
        scratch_types=[
            pltpu.VMEM((chunks_per_w, CHUNK), jnp.int32),
            pltpu.VMEM((2, GRP, H), jnp.float32),
            pltpu.SemaphoreType.DMA,
        ],
    )
    def gather_kernel(gidx_hbm, p_hbm, outa_hbm, outb_hbm, idx_v, rows_v, gsem):
        sid = lax.axis_index("s")
        wid = sid * NC + lax.axis_index("c")
        base_row = lax.rem(wid, NW // 2) * b_per_w
        pltpu.sync_copy(gidx_hbm.at[wid], idx_v)

        def body(g, carry):
            buf = lax.rem(g, 2)
            copies = []
            for c in range(CHUNKS_PER_GRP):
                cp = pltpu.async_copy(
                    p_hbm.at[idx_v.at[g * CHUNKS_PER_GRP + c]],
                    rows_v.at[buf, pl.ds(c * CHUNK, CHUNK)],
                    gsem)
                copies.append(cp)
            for cp in copies:
                cp.wait()
            dst = pl.ds(base_row + g * GRP, GRP)

            @pl.when(sid < half)
            def _():
                pltpu.sync_copy(rows_v.at[buf], outa_hbm.at[dst])

            @pl.when(sid >= half)
            def _():
                pltpu.sync_copy(rows_v.at[buf], outb_hbm.at[dst])

            return carry

        lax.fori_loop(0, grps, body, 0)

    return gather_kernel


def kernel(sent1, sent2, table, W, b):
    batch, seq = sent1.shape
    V, D = table.shape
    H = W.shape[1]
    g1 = sent1.transpose(1, 0).reshape(-1)
    g2 = sent2.transpose(1, 0).reshape(-1)
    gidx = jnp.concatenate([g1, g2])
    B = gidx.shape[0]
    gidx = gidx.reshape(NW, B // (NW * CHUNK), CHUNK)

    P = _project_table(table.T, W, b.reshape(1, H))
    outa, outb = _make_gather(B, H)(gidx, P)
    a = outa.reshape(seq, batch, H).transpose(1, 0, 2)
    bb = outb.reshape(seq, batch, H).transpose(1, 0, 2)
    return a, bb


# speedup vs baseline: 4.0404x; 1.0674x over previous
import functools

import jax
import jax.numpy as jnp
from jax import lax
from jax.experimental import pallas as pl
from jax.experimental.pallas import tpu as pltpu
from jax.experimental.pallas import tpu_sc as plsc

NC = 2
NS = 16
NW = NC * NS

CHUNK = 128
NBUF = 5

PROJ_BLK = 16384


def _project_table(tT, W, b2):
    D, V = tT.shape
    H = W.shape[1]
    grid = (pl.cdiv(V, PROJ_BLK),)

    def body(t_ref, w_ref, b_ref, p_ref):
        p_ref[...] = lax.dot_general(
            t_ref[...], w_ref[...],
            dimension_numbers=(((0,), (0,)), ((), ())),
            preferred_element_type=jnp.float32) + b_ref[...]

    return pl.pallas_call(
        body,
        grid=grid,
        in_specs=[
            pl.BlockSpec((D, PROJ_BLK), lambda i: (0, i)),
            pl.BlockSpec(W.shape, lambda i: (0, 0)),
            pl.BlockSpec((1, H), lambda i: (0, 0)),
        ],
        out_specs=pl.BlockSpec((PROJ_BLK, H), lambda i: (i, 0)),
        out_shape=jax.ShapeDtypeStruct((V, H), jnp.float32),
    )(tT, W, b2)


def _make_gather(Bs, H):
    bpw = Bs // NW
    cpw = bpw // CHUNK
    outer_iters = cpw // NBUF
    mesh = plsc.VectorSubcoreMesh(
        core_axis_name="c", subcore_axis_name="s",
        num_cores=NC, num_subcores=NS)

    @functools.partial(
        pl.kernel,
        mesh=mesh,
        out_type=jax.ShapeDtypeStruct((Bs, H), jnp.float32),
        scratch_types=[
            pltpu.VMEM((cpw, CHUNK), jnp.int32),
            pltpu.VMEM((NBUF, CHUNK, H), jnp.float32),
        ] + [pltpu.SemaphoreType.DMA] * (2 * NBUF),
    )
    def gather_kernel(gidx_hbm, p_hbm, out_hbm, idx_v, rows_v, *sems):
        gsem, osem = sems[:NBUF], sems[NBUF:]
        wid = lax.axis_index("s") * NC + lax.axis_index("c")
        base_row = wid * bpw
        pltpu.sync_copy(gidx_hbm.at[wid], idx_v)

        def gfire(i, b):
            pltpu.async_copy(p_hbm.at[idx_v.at[i]], rows_v.at[b], gsem[b])

        def gwait(b):
            pltpu.make_async_copy(out_hbm.at[pl.ds(0, CHUNK)],
                                  rows_v.at[b], gsem[b]).wait()

        def ofire(i, b):
            pltpu.async_copy(rows_v.at[b],
                             out_hbm.at[pl.ds(base_row + i * CHUNK, CHUNK)],
                             osem[b])

        def owait(b):
            pltpu.make_async_copy(rows_v.at[b],
                                  out_hbm.at[pl.ds(0, CHUNK)], osem[b]).wait()

        for b in range(NBUF - 1):
            gfire(b, b)
        for b in range(NBUF):
            gwait(b)
            ofire(b, b)
            bb = (b + NBUF - 1) % NBUF
            if b >= 1:
                owait(bb)
            gfire(b + NBUF - 1, bb)

        def outer(oo, carry):
            for b in range(NBUF):
                i = oo * NBUF + b
                gwait(b)
                ofire(i, b)
                bb = (b + NBUF - 1) % NBUF
                owait(bb)
                gfire(i + NBUF - 1, bb)
            return carry

        lax.fori_loop(1, outer_iters - 1, outer, 0)
        for b in range(NBUF):
            i = (outer_iters - 1) * NBUF + b
            gwait(b)
            ofire(i, b)
            if b == 0:
                bb = (b + NBUF - 1) % NBUF
                owait(bb)
                gfire(i + NBUF - 1, bb)
        for b in range(NBUF):
            owait(b)

    return gather_kernel


def kernel(sent1, sent2, table, W, b):
    batch, seq = sent1.shape
    V, D = table.shape
    H = W.shape[1]
    P = _project_table(table.T, W, b.reshape(1, H))

    gather = _make_gather(batch * seq, H)

    def one(sent):
        g = sent.transpose(1, 0).reshape(NW, (batch * seq) // (NW * CHUNK),
                                         CHUNK)
        out = gather(g, P)
        return out.reshape(seq, batch, H).transpose(1, 0, 2)

    return one(sent1), one(sent2)


# speedup vs baseline: 4.0489x; 1.0021x over previous
import functools

import jax
import jax.numpy as jnp
from jax import lax
from jax.experimental import pallas as pl
from jax.experimental.pallas import tpu as pltpu
from jax.experimental.pallas import tpu_sc as plsc

NC = 2
NS = 16
NW = NC * NS

CHUNK = 64
NBUF = 10

PROJ_BLK = 16384


def _project_table(tT, W, b2):
    D, V = tT.shape
    H = W.shape[1]
    grid = (pl.cdiv(V, PROJ_BLK),)

    def body(t_ref, w_ref, b_ref, p_ref):
        p_ref[...] = lax.dot_general(
            t_ref[...], w_ref[...],
            dimension_numbers=(((0,), (0,)), ((), ())),
            preferred_element_type=jnp.float32) + b_ref[...]

    return pl.pallas_call(
        body,
        grid=grid,
        in_specs=[
            pl.BlockSpec((D, PROJ_BLK), lambda i: (0, i)),
            pl.BlockSpec(W.shape, lambda i: (0, 0)),
            pl.BlockSpec((1, H), lambda i: (0, 0)),
        ],
        out_specs=pl.BlockSpec((PROJ_BLK, H), lambda i: (i, 0)),
        out_shape=jax.ShapeDtypeStruct((V, H), jnp.float32),
    )(tT, W, b2)


def _make_gather(Bs, H):
    bpw = Bs // NW
    cpw = bpw // CHUNK
    outer_iters = cpw // NBUF
    mesh = plsc.VectorSubcoreMesh(
        core_axis_name="c", subcore_axis_name="s",
        num_cores=NC, num_subcores=NS)

    @functools.partial(
        pl.kernel,
        mesh=mesh,
        out_type=jax.ShapeDtypeStruct((Bs, H), jnp.float32),
        scratch_types=[
            pltpu.VMEM((cpw, CHUNK), jnp.int32),
            pltpu.VMEM((NBUF, CHUNK, H), jnp.float32),
        ] + [pltpu.SemaphoreType.DMA] * (2 * NBUF),
    )
    def gather_kernel(gidx_hbm, p_hbm, out_hbm, idx_v, rows_v, *sems):
        gsem, osem = sems[:NBUF], sems[NBUF:]
        wid = lax.axis_index("s") * NC + lax.axis_index("c")
        base_row = wid * bpw
        pltpu.sync_copy(gidx_hbm.at[wid], idx_v)

        def gfire(i, b):
            pltpu.async_copy(p_hbm.at[idx_v.at[i]], rows_v.at[b], gsem[b])

        def gwait(b):
            pltpu.make_async_copy(out_hbm.at[pl.ds(0, CHUNK)],
                                  rows_v.at[b], gsem[b]).wait()

        def ofire(i, b):
            pltpu.async_copy(rows_v.at[b],
                             out_hbm.at[pl.ds(base_row + i * CHUNK, CHUNK)],
                             osem[b])

        def owait(b):
            pltpu.make_async_copy(rows_v.at[b],
                                  out_hbm.at[pl.ds(0, CHUNK)], osem[b]).wait()

        for b in range(NBUF - 1):
            gfire(b, b)
        for b in range(NBUF):
            gwait(b)
            ofire(b, b)
            bb = (b + NBUF - 1) % NBUF
            if b >= 1:
                owait(bb)
            gfire(b + NBUF - 1, bb)

        def outer(oo, carry):
            for b in range(NBUF):
                i = oo * NBUF + b
                gwait(b)
                ofire(i, b)
                bb = (b + NBUF - 1) % NBUF
                owait(bb)
                gfire(i + NBUF - 1, bb)
            return carry

        lax.fori_loop(1, outer_iters - 1, outer, 0)
        for b in range(NBUF):
            i = (outer_iters - 1) * NBUF + b
            gwait(b)
            ofire(i, b)
            if b == 0:
                bb = (b + NBUF - 1) % NBUF
                owait(bb)
                gfire(i + NBUF - 1, bb)
        for b in range(NBUF):
            owait(b)

    return gather_kernel


def kernel(sent1, sent2, table, W, b):
    batch, seq = sent1.shape
    V, D = table.shape
    H = W.shape[1]
    P = _project_table(table.T, W, b.reshape(1, H))

    gather = _make_gather(batch * seq, H)

    def one(sent):
        g = sent.transpose(1, 0).reshape(NW, (batch * seq) // (NW * CHUNK),
                                         CHUNK)
        out = gather(g, P)
        return out.reshape(seq, batch, H).transpose(1, 0, 2)

    return one(sent1), one(sent2)


# speedup vs baseline: 4.1026x; 1.0133x over previous
import functools

import jax
import jax.numpy as jnp
from jax import lax
from jax.experimental import pallas as pl
from jax.experimental.pallas import tpu as pltpu
from jax.experimental.pallas import tpu_sc as plsc

NC = 2
NS = 16
NW = NC * NS

CHUNK = 128
NBUF = 5

PROJ_BLK = 32768


def _project_table(tT, W, b2):
    D, V = tT.shape
    H = W.shape[1]
    grid = (pl.cdiv(V, PROJ_BLK),)

    def body(t_ref, w_ref, b_ref, p_ref):
        p_ref[...] = lax.dot_general(
            t_ref[...], w_ref[...],
            dimension_numbers=(((0,), (0,)), ((), ())),
            preferred_element_type=jnp.float32) + b_ref[...]

    return pl.pallas_call(
        body,
        grid=grid,
        in_specs=[
            pl.BlockSpec((D, PROJ_BLK), lambda i: (0, i)),
            pl.BlockSpec(W.shape, lambda i: (0, 0)),
            pl.BlockSpec((1, H), lambda i: (0, 0)),
        ],
        out_specs=pl.BlockSpec((PROJ_BLK, H), lambda i: (i, 0)),
        out_shape=jax.ShapeDtypeStruct((V, H), jnp.float32),
    )(tT, W, b2)


def _make_gather(Bs, H):
    bpw = Bs // NW
    cpw = bpw // CHUNK
    outer_iters = cpw // NBUF
    mesh = plsc.VectorSubcoreMesh(
        core_axis_name="c", subcore_axis_name="s",
        num_cores=NC, num_subcores=NS)

    @functools.partial(
        pl.kernel,
        mesh=mesh,
        out_type=jax.ShapeDtypeStruct((Bs, H), jnp.float32),
        scratch_types=[
            pltpu.VMEM((cpw, CHUNK), jnp.int32),
            pltpu.VMEM((NBUF, CHUNK, H), jnp.float32),
        ] + [pltpu.SemaphoreType.DMA] * (2 * NBUF),
    )
    def gather_kernel(gidx_hbm, p_hbm, out_hbm, idx_v, rows_v, *sems):
        gsem, osem = sems[:NBUF], sems[NBUF:]
        wid = lax.axis_index("s") * NC + lax.axis_index("c")
        base_row = wid * bpw
        pltpu.sync_copy(gidx_hbm.at[wid], idx_v)

        def gfire(i, b):
            pltpu.async_copy(p_hbm.at[idx_v.at[i]], rows_v.at[b], gsem[b])

        def gwait(b):
            pltpu.make_async_copy(out_hbm.at[pl.ds(0, CHUNK)],
                                  rows_v.at[b], gsem[b]).wait()

        def ofire(i, b):
            pltpu.async_copy(rows_v.at[b],
                             out_hbm.at[pl.ds(base_row + i * CHUNK, CHUNK)],
                             osem[b])

        def owait(b):
            pltpu.make_async_copy(rows_v.at[b],
                                  out_hbm.at[pl.ds(0, CHUNK)], osem[b]).wait()

        for b in range(NBUF - 1):
            gfire(b, b)
        for b in range(NBUF):
            gwait(b)
            ofire(b, b)
            bb = (b + NBUF - 1) % NBUF
            if b >= 1:
                owait(bb)
            gfire(b + NBUF - 1, bb)

        def outer(oo, carry):
            for b in range(NBUF):
                i = oo * NBUF + b
                gwait(b)
                ofire(i, b)
                bb = (b + NBUF - 1) % NBUF
                owait(bb)
                gfire(i + NBUF - 1, bb)
            return carry

        lax.fori_loop(1, outer_iters - 1, outer, 0)
        for b in range(NBUF):
            i = (outer_iters - 1) * NBUF + b
            gwait(b)
            ofire(i, b)
            if b == 0:
                bb = (b + NBUF - 1) % NBUF
                owait(bb)
                gfire(i + NBUF - 1, bb)
        for b in range(NBUF):
            owait(b)

    return gather_kernel


def kernel(sent1, sent2, table, W, b):
    batch, seq = sent1.shape
    V, D = table.shape
    H = W.shape[1]
    P = _project_table(table.T, W, b.reshape(1, H))

    gather = _make_gather(batch * seq, H)

    def one(sent):
        g = sent.transpose(1, 0).reshape(NW, (batch * seq) // (NW * CHUNK),
                                         CHUNK)
        out = gather(g, P)
        return out.reshape(seq, batch, H).transpose(1, 0, 2)

    return one(sent1), one(sent2)
